# baseline (device time: 315858 ns/iter reference)
import jax
import jax.numpy as jnp
from jax import lax
from jax.experimental import pallas as pl
from jax.experimental.pallas import tpu as pltpu

N_DEV = 4


def kernel(x, w_mat):
    m_per, k = x.shape
    _, n_loc = w_mat.shape

    x = x.astype(jnp.bfloat16)
    w_mat = w_mat.astype(jnp.bfloat16)

    def body(x_ref, w_ref, out_ref, comm_ref, amax_ref,
             send_sems, recv_sems, amax_send_sems, amax_recv_sems):
        my = lax.axis_index("i")
        left = (my - 1) % N_DEV
        right = (my + 1) % N_DEV

        barrier_sem = pltpu.get_barrier_semaphore()
        for nbr in (left, right):
            pl.semaphore_signal(
                barrier_sem, inc=1,
                device_id=(nbr,), device_id_type=pl.DeviceIdType.MESH,
            )
        pl.semaphore_wait(barrier_sem, 2)

        out_ref[pl.ds(my * m_per, m_per), :] = jnp.dot(
            x_ref[...], w_ref[...], preferred_element_type=jnp.float32
        )

        for h in range(N_DEV - 1):
            src = x_ref if h == 0 else comm_ref.at[h - 1]
            rdma = pltpu.make_async_remote_copy(
                src_ref=src,
                dst_ref=comm_ref.at[h],
                send_sem=send_sems.at[h],
                recv_sem=recv_sems.at[h],
                device_id=(right,),
                device_id_type=pl.DeviceIdType.MESH,
            )
            rdma.start()
            rdma.wait()
            origin = (my - h - 1) % N_DEV
            out_ref[pl.ds(origin * m_per, m_per), :] = jnp.dot(
                comm_ref[h], w_ref[...], preferred_element_type=jnp.float32
            )

        local_amax = jnp.max(jnp.abs(out_ref[...]))
        amax_ref[0, :, :] = jnp.full((8, 128), local_amax, jnp.float32)
        amax_rdmas = []
        for d in range(1, N_DEV):
            target = (my + d) % N_DEV
            rdma = pltpu.make_async_remote_copy(
                src_ref=amax_ref.at[0],
                dst_ref=amax_ref.at[d],
                send_sem=amax_send_sems.at[d - 1],
                recv_sem=amax_recv_sems.at[d - 1],
                device_id=(target,),
                device_id_type=pl.DeviceIdType.MESH,
            )
            rdma.start()
            amax_rdmas.append(rdma)
        for rdma in amax_rdmas:
            rdma.wait()

        amax = jnp.max(amax_ref[:, 0, 0])
        scale = amax / 127.0
        q = jnp.clip(jnp.round(out_ref[...] / scale), -127.0, 127.0)
        out_ref[...] = q * scale

    return pl.pallas_call(
        body,
        out_shape=jax.ShapeDtypeStruct((N_DEV * m_per, n_loc), jnp.float32),
        in_specs=[
            pl.BlockSpec(memory_space=pltpu.VMEM),
            pl.BlockSpec(memory_space=pltpu.VMEM),
        ],
        out_specs=pl.BlockSpec(memory_space=pltpu.VMEM),
        scratch_shapes=[
            pltpu.VMEM((N_DEV - 1, m_per, k), jnp.bfloat16),
            pltpu.VMEM((N_DEV, 8, 128), jnp.float32),
            pltpu.SemaphoreType.DMA((N_DEV - 1,)),
            pltpu.SemaphoreType.DMA((N_DEV - 1,)),
            pltpu.SemaphoreType.DMA((N_DEV - 1,)),
            pltpu.SemaphoreType.DMA((N_DEV - 1,)),
        ],
        compiler_params=pltpu.CompilerParams(collective_id=0),
    )(x, w_mat)


# device time: 165823 ns/iter; 1.9048x vs baseline; 1.9048x over previous
import jax
import jax.numpy as jnp
from jax import lax
from jax.experimental import pallas as pl
from jax.experimental.pallas import tpu as pltpu

N_DEV = 4


def kernel(x, w_mat):
    m_per, k = x.shape
    _, n_loc = w_mat.shape
    half = m_per // 2

    x = x.astype(jnp.bfloat16)
    w_mat = w_mat.astype(jnp.bfloat16)

    def body(x_ref, w_ref, out_ref, comm_cw, comm_ccw, amax_ref,
             cw_send, cw_recv, ccw_send, ccw_recv,
             amax_send_sems, amax_recv_sems):
        my = lax.axis_index("i")
        left = (my - 1) % N_DEV
        right = (my + 1) % N_DEV

        barrier_sem = pltpu.get_barrier_semaphore()
        for nbr in (left, right):
            pl.semaphore_signal(
                barrier_sem, inc=1,
                device_id=(nbr,), device_id_type=pl.DeviceIdType.MESH,
            )
        pl.semaphore_wait(barrier_sem, 2)

        prev_cw = pltpu.make_async_remote_copy(
            src_ref=x_ref.at[pl.ds(0, half)],
            dst_ref=comm_cw.at[0],
            send_sem=cw_send.at[0],
            recv_sem=cw_recv.at[0],
            device_id=(right,),
            device_id_type=pl.DeviceIdType.MESH,
        )
        prev_ccw = pltpu.make_async_remote_copy(
            src_ref=x_ref.at[pl.ds(half, half)],
            dst_ref=comm_ccw.at[0],
            send_sem=ccw_send.at[0],
            recv_sem=ccw_recv.at[0],
            device_id=(left,),
            device_id_type=pl.DeviceIdType.MESH,
        )
        prev_cw.start()
        prev_ccw.start()

        y_local = jnp.dot(x_ref[...], w_ref[...],
                          preferred_element_type=jnp.float32)
        out_ref[pl.ds(my * m_per, m_per), :] = y_local
        amax_run = jnp.max(jnp.abs(y_local))

        for h in range(N_DEV - 1):
            prev_cw.wait_recv()
            prev_ccw.wait_recv()
            prev_cw.wait_send()
            prev_ccw.wait_send()
            if h < N_DEV - 2:
                prev_cw = pltpu.make_async_remote_copy(
                    src_ref=comm_cw.at[h],
                    dst_ref=comm_cw.at[h + 1],
                    send_sem=cw_send.at[h + 1],
                    recv_sem=cw_recv.at[h + 1],
                    device_id=(right,),
                    device_id_type=pl.DeviceIdType.MESH,
                )
                prev_ccw = pltpu.make_async_remote_copy(
                    src_ref=comm_ccw.at[h],
                    dst_ref=comm_ccw.at[h + 1],
                    send_sem=ccw_send.at[h + 1],
                    recv_sem=ccw_recv.at[h + 1],
                    device_id=(left,),
                    device_id_type=pl.DeviceIdType.MESH,
                )
                prev_cw.start()
                prev_ccw.start()
            o_cw = (my - h - 1) % N_DEV
            o_ccw = (my + h + 1) % N_DEV
            y_cw = jnp.dot(comm_cw[h], w_ref[...],
                           preferred_element_type=jnp.float32)
            out_ref[pl.ds(o_cw * m_per, half), :] = y_cw
            y_ccw = jnp.dot(comm_ccw[h], w_ref[...],
                            preferred_element_type=jnp.float32)
            out_ref[pl.ds(o_ccw * m_per + half, half), :] = y_ccw
            amax_run = jnp.maximum(
                amax_run,
                jnp.maximum(jnp.max(jnp.abs(y_cw)), jnp.max(jnp.abs(y_ccw))),
            )

        amax_ref[0, :, :] = jnp.full((8, 128), amax_run, jnp.float32)
        amax_rdmas = []
        for d in range(1, N_DEV):
            target = (my + d) % N_DEV
            rdma = pltpu.make_async_remote_copy(
                src_ref=amax_ref.at[0],
                dst_ref=amax_ref.at[d],
                send_sem=amax_send_sems.at[d - 1],
                recv_sem=amax_recv_sems.at[d - 1],
                device_id=(target,),
                device_id_type=pl.DeviceIdType.MESH,
            )
            rdma.start()
            amax_rdmas.append(rdma)
        for rdma in amax_rdmas:
            rdma.wait()

        amax = jnp.max(amax_ref[:, 0, 0])
        scale = amax / 127.0
        q = jnp.clip(jnp.round(out_ref[...] / scale), -127.0, 127.0)
        out_ref[...] = q * scale

    return pl.pallas_call(
        body,
        out_shape=jax.ShapeDtypeStruct((N_DEV * m_per, n_loc), jnp.float32),
        in_specs=[
            pl.BlockSpec(memory_space=pltpu.VMEM),
            pl.BlockSpec(memory_space=pltpu.VMEM),
        ],
        out_specs=pl.BlockSpec(memory_space=pltpu.VMEM),
        scratch_shapes=[
            pltpu.VMEM((N_DEV - 1, half, k), jnp.bfloat16),
            pltpu.VMEM((N_DEV - 1, half, k), jnp.bfloat16),
            pltpu.VMEM((N_DEV, 8, 128), jnp.float32),
            pltpu.SemaphoreType.DMA((N_DEV - 1,)),
            pltpu.SemaphoreType.DMA((N_DEV - 1,)),
            pltpu.SemaphoreType.DMA((N_DEV - 1,)),
            pltpu.SemaphoreType.DMA((N_DEV - 1,)),
            pltpu.SemaphoreType.DMA((N_DEV - 1,)),
            pltpu.SemaphoreType.DMA((N_DEV - 1,)),
        ],
        compiler_params=pltpu.CompilerParams(collective_id=0),
    )(x, w_mat)
